# baseline (speedup 1.0000x reference)
import functools

import jax
import jax.numpy as jnp
from jax import lax
from jax.experimental import pallas as pl
from jax.experimental.pallas import tpu as pltpu
from jax.experimental.pallas import tpu_sc as plsc

N = 2097152
NC, NS = 2, 16
NW = NC * NS
SPAN = N // NW
CH = 16384
NCHUNK = SPAN // CH
GROUPS = CH // 64

_mesh = plsc.VectorSubcoreMesh(
    core_axis_name="c", subcore_axis_name="s", num_cores=NC, num_subcores=NS)


@functools.partial(
    pl.kernel,
    out_type=(
        jax.ShapeDtypeStruct((N,), jnp.float32),
        jax.ShapeDtypeStruct((N // 4,), jnp.int32),
    ),
    mesh=_mesh,
    scratch_types=[
        pltpu.VMEM((CH,), jnp.float32),
        pltpu.VMEM((CH,), jnp.float32),
        pltpu.VMEM((CH // 4,), jnp.int32),
    ],
    compiler_params=pltpu.CompilerParams(needs_layout_passes=False),
)
def _vq(x_hbm, vals_hbm, words_hbm, x_v, vals_v, words_v):
    wid = lax.axis_index("s") * NC + lax.axis_index("c")
    base = wid * SPAN
    iota4 = lax.iota(jnp.int32, 16) * 4

    def chunk_body(ci, _):
        off = pl.multiple_of(base + ci * CH, CH)
        woff = pl.multiple_of(off // 4, CH // 4)
        pltpu.sync_copy(x_hbm.at[pl.ds(off, CH)], x_v)

        def group(g, _):
            gb = g * 64
            word = jnp.zeros((16,), jnp.int32)
            for c in range(4):
                idxv = iota4 + (gb + c)
                xv = plsc.load_gather(x_v, [idxv])
                t = jnp.clip(xv + 7.0, 0.0, 15.0)
                ti = t.astype(jnp.int32)
                ti = jnp.where(t > ti.astype(jnp.float32), ti + 1, ti)
                plsc.store_scatter(vals_v, [idxv], ti.astype(jnp.float32) - 7.5)
                word = word | (ti << (8 * c))
            words_v[pl.ds(g * 16, 16)] = word
            return 0

        lax.fori_loop(0, GROUPS, group, 0)
        pltpu.sync_copy(vals_v, vals_hbm.at[pl.ds(off, CH)])
        pltpu.sync_copy(words_v, words_hbm.at[pl.ds(woff, CH // 4)])
        return 0

    lax.fori_loop(0, NCHUNK, chunk_body, 0)


def kernel(X, grid, grid_norm):
    vals, words = _vq(X.reshape(N))
    idx = lax.bitcast_convert_type(words, jnp.uint8).reshape(N)
    return (vals.reshape(N, 1), idx)


# speedup vs baseline: 11.6421x; 11.6421x over previous
import functools

import jax
import jax.numpy as jnp
from jax import lax
from jax.experimental import pallas as pl
from jax.experimental.pallas import tpu as pltpu
from jax.experimental.pallas import tpu_sc as plsc

N = 2097152
R, C = 16384, 128
NC, NS = 2, 16
NW = NC * NS
RPW = R // NW
CR = 128
NCHUNK = RPW // CR

_mesh = plsc.VectorSubcoreMesh(
    core_axis_name="c", subcore_axis_name="s", num_cores=NC, num_subcores=NS)


@functools.partial(
    pl.kernel,
    out_type=jax.ShapeDtypeStruct((R, C), jnp.float32),
    mesh=_mesh,
    scratch_types=[
        pltpu.VMEM((CR, C), jnp.float32),
        pltpu.VMEM((CR, C), jnp.float32),
    ],
    compiler_params=pltpu.CompilerParams(needs_layout_passes=False),
)
def _vq_vals(x_hbm, vals_hbm, x_v, vals_v):
    wid = lax.axis_index("s") * NC + lax.axis_index("c")
    rbase = wid * RPW

    def chunk_body(ci, _):
        roff = pl.multiple_of(rbase + ci * CR, CR)
        pltpu.sync_copy(x_hbm.at[pl.ds(roff, CR), :], x_v)

        @plsc.parallel_loop(0, CR)
        def row_body(r):
            for c in range(C // 16):
                xv = x_v[r, pl.ds(c * 16, 16)]
                u = jnp.clip(8.0 - xv, 0.0, 15.0).astype(jnp.int32)
                vals_v[r, pl.ds(c * 16, 16)] = 7.5 - u.astype(jnp.float32)

        pltpu.sync_copy(vals_v, vals_hbm.at[pl.ds(roff, CR), :])
        return 0

    lax.fori_loop(0, NCHUNK, chunk_body, 0)


def _idx_body(x_ref, idx_ref):
    u = jnp.clip(8.0 - x_ref[...], 0.0, 15.0).astype(jnp.int32)
    idx_ref[...] = (15 - u).astype(jnp.uint8)


_BR = 1024

_vq_idx = pl.pallas_call(
    _idx_body,
    grid=(R // _BR,),
    in_specs=[pl.BlockSpec((_BR, C), lambda i: (i, 0))],
    out_specs=pl.BlockSpec((_BR, C), lambda i: (i, 0)),
    out_shape=jax.ShapeDtypeStruct((R, C), jnp.uint8),
    compiler_params=pltpu.CompilerParams(
        dimension_semantics=("arbitrary",)),
)


def kernel(X, grid, grid_norm):
    x2d = X.reshape(R, C)
    vals2d = _vq_vals(x2d)
    idx2d = _vq_idx(x2d)
    return (vals2d.reshape(N, 1), idx2d.reshape(N))


# speedup vs baseline: 13.1698x; 1.1312x over previous
import functools

import jax
import jax.numpy as jnp
from jax import lax
from jax.experimental import pallas as pl
from jax.experimental.pallas import tpu as pltpu
from jax.experimental.pallas import tpu_sc as plsc

N = 2097152
R, C = 16384, 128
NC, NS = 2, 16
NW = NC * NS
RPW = R // NW
CR = 128
NCHUNK = RPW // CR

_mesh = plsc.VectorSubcoreMesh(
    core_axis_name="c", subcore_axis_name="s", num_cores=NC, num_subcores=NS)


@functools.partial(
    pl.kernel,
    out_type=jax.ShapeDtypeStruct((R, C), jnp.float32),
    mesh=_mesh,
    scratch_types=[
        pltpu.VMEM((CR, C), jnp.float32),
        pltpu.VMEM((CR, C), jnp.float32),
        pltpu.VMEM((CR, C), jnp.float32),
        pltpu.VMEM((CR, C), jnp.float32),
        pltpu.SemaphoreType.DMA,
        pltpu.SemaphoreType.DMA,
        pltpu.SemaphoreType.DMA,
        pltpu.SemaphoreType.DMA,
    ],
    compiler_params=pltpu.CompilerParams(needs_layout_passes=False),
)
def _vq_vals(x_hbm, vals_hbm, x_v0, x_v1, o_v0, o_v1,
             si0, si1, so0, so1):
    wid = lax.axis_index("s") * NC + lax.axis_index("c")
    rbase = wid * RPW
    x_bufs, o_bufs = (x_v0, x_v1), (o_v0, o_v1)
    si, so = (si0, si1), (so0, so1)

    def row_slice(ci):
        return pl.ds(pl.multiple_of(rbase + ci * CR, CR), CR)

    def start_in(ci):
        return pltpu.async_copy(
            x_hbm.at[row_slice(ci), :], x_bufs[ci % 2], si[ci % 2])

    in_copies = {0: start_in(0)}
    out_copies = {}
    for ci in range(NCHUNK):
        if ci + 1 < NCHUNK:
            in_copies[ci + 1] = start_in(ci + 1)
        in_copies[ci].wait()
        if ci >= 2:
            out_copies[ci - 2].wait()
        x_v, vals_v = x_bufs[ci % 2], o_bufs[ci % 2]

        @plsc.parallel_loop(0, CR)
        def row_body(r):
            for c in range(C // 16):
                xv = x_v[r, pl.ds(c * 16, 16)]
                u = jnp.clip(8.0 - xv, 0.0, 15.0).astype(jnp.int32)
                vals_v[r, pl.ds(c * 16, 16)] = 7.5 - u.astype(jnp.float32)

        out_copies[ci] = pltpu.async_copy(
            vals_v, vals_hbm.at[row_slice(ci), :], so[ci % 2])
    out_copies[NCHUNK - 2].wait()
    out_copies[NCHUNK - 1].wait()


def _idx_body(x_ref, idx_ref):
    u = jnp.clip(8.0 - x_ref[...], 0.0, 15.0).astype(jnp.int32)
    idx_ref[...] = (15 - u).astype(jnp.uint8)


_BR = 1024

_vq_idx = pl.pallas_call(
    _idx_body,
    grid=(R // _BR,),
    in_specs=[pl.BlockSpec((_BR, C), lambda i: (i, 0))],
    out_specs=pl.BlockSpec((_BR, C), lambda i: (i, 0)),
    out_shape=jax.ShapeDtypeStruct((R, C), jnp.uint8),
    compiler_params=pltpu.CompilerParams(
        dimension_semantics=("arbitrary",)),
)


def kernel(X, grid, grid_norm):
    x2d = X.reshape(R, C)
    vals2d = _vq_vals(x2d)
    idx2d = _vq_idx(x2d)
    return (vals2d.reshape(N, 1), idx2d.reshape(N))
